# baseline (speedup 1.0000x reference)
import functools

import jax
import jax.numpy as jnp
from jax import lax
from jax.experimental import pallas as pl
from jax.experimental.pallas import tpu as pltpu
from jax.experimental.pallas import tpu_sc as plsc

_NC = 2
_NS = 16
_CHUNK = 80


def _make_sc_agg(N, E, D, chunk, nbuf, with_deg):
    NW = _NC * _NS
    epw = E // NW
    nchunk = epw // chunk
    nfull = nchunk // nbuf
    rpt = (N // _NS) // 8 * 8
    tail = N - _NS * rpt

    mesh = plsc.VectorSubcoreMesh(core_axis_name="c", subcore_axis_name="s")

    out_type = [jax.ShapeDtypeStruct((_NC, N, D), jnp.float32)]
    scratch = [
        pltpu.VMEM((nchunk, chunk), jnp.int32),
        pltpu.VMEM((nchunk, chunk), jnp.int32),
        [pltpu.VMEM((chunk, D), jnp.float32) for _ in range(nbuf)],
        [pltpu.SemaphoreType.DMA for _ in range(nbuf)],
        [pltpu.SemaphoreType.DMA for _ in range(nbuf)],
        pltpu.VMEM_SHARED((N, D), jnp.float32),
    ]
    if with_deg:
        out_type.append(jax.ShapeDtypeStruct((_NC, N, 16), jnp.float32))
        scratch += [
            pltpu.VMEM((chunk, 16), jnp.float32),
            pltpu.SemaphoreType.DMA,
            pltpu.VMEM_SHARED((N, 16), jnp.float32),
        ]

    def body(p_hbm, src_hbm, dst_hbm, z_hbm, z16_hbm, *refs):
        if with_deg:
            (agg_out, deg_out, idx_s, idx_d, rows, gsem, ssem, agg_sp,
             ones, dsem, deg_sp) = refs
        else:
            agg_out, idx_s, idx_d, rows, gsem, ssem, agg_sp = refs
        c = lax.axis_index("c")
        s = lax.axis_index("s")
        wid = c * _NS + s

        pltpu.sync_copy(src_hbm.at[pl.ds(wid * nchunk, nchunk)], idx_s)
        pltpu.sync_copy(dst_hbm.at[pl.ds(wid * nchunk, nchunk)], idx_d)

        r0 = s * rpt
        pltpu.sync_copy(z_hbm.at[pl.ds(r0, rpt)], agg_sp.at[pl.ds(r0, rpt)])
        if with_deg:
            pltpu.sync_copy(z16_hbm.at[pl.ds(r0, rpt)],
                            deg_sp.at[pl.ds(r0, rpt)])
            for j in range(chunk):
                ones[j, :] = jnp.ones((16,), jnp.float32)
        if tail:
            t0 = _NS * rpt

            @pl.when(s == _NS - 1)
            def _():
                pltpu.sync_copy(z_hbm.at[pl.ds(t0, tail)],
                                agg_sp.at[pl.ds(t0, tail)])
                if with_deg:
                    pltpu.sync_copy(z16_hbm.at[pl.ds(t0, tail)],
                                    deg_sp.at[pl.ds(t0, tail)])
        plsc.subcore_barrier()

        def g_start(i, b):
            pltpu.async_copy(p_hbm.at[idx_s.at[i]], rows[b], gsem[b])

        def g_wait(b):
            pltpu.make_async_copy(p_hbm.at[idx_s.at[0]], rows[b],
                                  gsem[b]).wait()

        def s_start(i, b):
            pltpu.async_copy(rows[b], agg_sp.at[idx_d.at[i]], ssem[b],
                             add=True)
            if with_deg:
                pltpu.async_copy(ones, deg_sp.at[idx_d.at[i]], dsem,
                                 add=True)

        def s_wait(b):
            pltpu.make_async_copy(rows[b], agg_sp.at[idx_d.at[0]],
                                  ssem[b]).wait()
            if with_deg:
                pltpu.make_async_copy(ones, deg_sp.at[idx_d.at[0]],
                                      dsem).wait()

        for b in range(nbuf):
            g_start(b, b)

        def round_body(r, carry):
            i0 = r * nbuf
            for b in range(nbuf):
                g_wait(b)
                s_start(i0 + b, b)
            for b in range(nbuf):
                i_next = i0 + nbuf + b
                s_wait(b)

                @pl.when(i_next < nchunk)
                def _():
                    g_start(i_next, b)
            return carry

        lax.fori_loop(0, nfull, round_body, 0)
        for b in range(nchunk - nfull * nbuf):
            g_wait(b)
            s_start(nfull * nbuf + b, b)
            s_wait(b)
        plsc.subcore_barrier()

        pltpu.sync_copy(agg_sp.at[pl.ds(r0, rpt)],
                        agg_out.at[c, pl.ds(r0, rpt)])
        if with_deg:
            pltpu.sync_copy(deg_sp.at[pl.ds(r0, rpt)],
                            deg_out.at[c, pl.ds(r0, rpt)])
        if tail:
            t0 = _NS * rpt

            @pl.when(s == _NS - 1)
            def _():
                pltpu.sync_copy(agg_sp.at[pl.ds(t0, tail)],
                                agg_out.at[c, pl.ds(t0, tail)])
                if with_deg:
                    pltpu.sync_copy(deg_sp.at[pl.ds(t0, tail)],
                                    deg_out.at[c, pl.ds(t0, tail)])

    return pl.kernel(body, out_type=out_type, mesh=mesh,
                     scratch_types=scratch,
                     compiler_params=pltpu.CompilerParams(
                         use_tc_tiling_on_sc=False))


def _mm_first(x_ref, ws_ref, wn_ref, ps_ref, pn_ref):
    x = x_ref[...]
    ps_ref[...] = jnp.dot(x, ws_ref[...], preferred_element_type=jnp.float32,
                          precision=lax.Precision.HIGHEST)
    pn_ref[...] = jnp.dot(x, wn_ref[...], preferred_element_type=jnp.float32,
                          precision=lax.Precision.HIGHEST)


def _combine(ps_ref, agg_ref, deg_ref, b_ref, h_ref):
    deg = deg_ref[0, :, 0:1] + deg_ref[1, :, 0:1]
    inv = 1.0 / jnp.maximum(deg, 1.0)
    h_ref[...] = ps_ref[...] + (agg_ref[0] + agg_ref[1]) * inv + b_ref[...]


def _combine_mm(ps_ref, agg_ref, deg_ref, b_ref, ws_ref, wn_ref,
                ps_out, pn_out):
    deg = deg_ref[0, :, 0:1] + deg_ref[1, :, 0:1]
    inv = 1.0 / jnp.maximum(deg, 1.0)
    h = ps_ref[...] + (agg_ref[0] + agg_ref[1]) * inv + b_ref[...]
    h = jnp.maximum(h, 0.0)
    ps_out[...] = jnp.dot(h, ws_ref[...], preferred_element_type=jnp.float32,
                          precision=lax.Precision.HIGHEST)
    pn_out[...] = jnp.dot(h, wn_ref[...], preferred_element_type=jnp.float32,
                          precision=lax.Precision.HIGHEST)


def _tc_first(x, ws, wn, blk):
    N, Din = x.shape
    Dout = ws.shape[1]
    return pl.pallas_call(
        _mm_first,
        grid=(N // blk,),
        in_specs=[
            pl.BlockSpec((blk, Din), lambda i: (i, 0)),
            pl.BlockSpec((Din, Dout), lambda i: (0, 0)),
            pl.BlockSpec((Din, Dout), lambda i: (0, 0)),
        ],
        out_specs=[
            pl.BlockSpec((blk, Dout), lambda i: (i, 0)),
            pl.BlockSpec((blk, Dout), lambda i: (i, 0)),
        ],
        out_shape=[jax.ShapeDtypeStruct((N, Dout), jnp.float32)] * 2,
    )(x, ws, wn)


def _tc_combine_mm(ps, agg, deg, b, ws, wn, blk):
    N, Dh = ps.shape
    Dout = ws.shape[1]
    return pl.pallas_call(
        _combine_mm,
        grid=(N // blk,),
        in_specs=[
            pl.BlockSpec((blk, Dh), lambda i: (i, 0)),
            pl.BlockSpec((2, blk, Dh), lambda i: (0, i, 0)),
            pl.BlockSpec((2, blk, 16), lambda i: (0, i, 0)),
            pl.BlockSpec((1, Dh), lambda i: (0, 0)),
            pl.BlockSpec((Dh, Dout), lambda i: (0, 0)),
            pl.BlockSpec((Dh, Dout), lambda i: (0, 0)),
        ],
        out_specs=[
            pl.BlockSpec((blk, Dout), lambda i: (i, 0)),
            pl.BlockSpec((blk, Dout), lambda i: (i, 0)),
        ],
        out_shape=[jax.ShapeDtypeStruct((N, Dout), jnp.float32)] * 2,
    )(ps, agg, deg, b, ws, wn)


def _tc_combine(ps, agg, deg, b, blk):
    N, Dh = ps.shape
    return pl.pallas_call(
        _combine,
        grid=(N // blk,),
        in_specs=[
            pl.BlockSpec((blk, Dh), lambda i: (i, 0)),
            pl.BlockSpec((2, blk, Dh), lambda i: (0, i, 0)),
            pl.BlockSpec((2, blk, 16), lambda i: (0, i, 0)),
            pl.BlockSpec((1, Dh), lambda i: (0, 0)),
        ],
        out_specs=pl.BlockSpec((blk, Dh), lambda i: (i, 0)),
        out_shape=jax.ShapeDtypeStruct((N, Dh), jnp.float32),
    )(ps, agg, deg, b)


@jax.jit
def kernel(x, edge_index, W_self0, W_neigh0, b0, W_self1, W_neigh1, b1,
           W_self2, W_neigh2, b2):
    N, _ = x.shape
    E = edge_index.shape[1]
    D_hid = W_self0.shape[1]
    D_out = W_self2.shape[1]
    blk = 1000

    c0 = 40
    c1 = 80
    src40 = edge_index[0].reshape(E // c0, c0)
    dst40 = edge_index[1].reshape(E // c0, c0)
    src80 = edge_index[0].reshape(E // c1, c1)
    dst80 = edge_index[1].reshape(E // c1, c1)
    z_h = jnp.zeros((N, D_hid), jnp.float32)
    z_o = jnp.zeros((N, D_out), jnp.float32)
    z16 = jnp.zeros((N, 16), jnp.float32)

    agg_hd = _make_sc_agg(N, E, D_hid, c0, 3, True)
    agg_h = _make_sc_agg(N, E, D_hid, c1, 3, False)
    agg_o = _make_sc_agg(N, E, D_out, c1, 3, False)

    ps0, pn0 = _tc_first(x, W_self0, W_neigh0, blk)
    agg0, deg = agg_hd(pn0, src40, dst40, z_h, z16)
    ps1, pn1 = _tc_combine_mm(ps0, agg0, deg, b0.reshape(1, -1),
                              W_self1, W_neigh1, blk)
    (agg1,) = agg_h(pn1, src80, dst80, z_h, z16)
    ps2, pn2 = _tc_combine_mm(ps1, agg1, deg, b1.reshape(1, -1),
                              W_self2, W_neigh2, blk)
    (agg2,) = agg_o(pn2, src80, dst80, z_o, z16)
    return _tc_combine(ps2, agg2, deg, b2.reshape(1, -1), blk)


# speedup vs baseline: 1.1261x; 1.1261x over previous
import functools

import jax
import jax.numpy as jnp
from jax import lax
from jax.experimental import pallas as pl
from jax.experimental.pallas import tpu as pltpu
from jax.experimental.pallas import tpu_sc as plsc

_NC = 2
_NS = 16
_CHUNK = 80


def _make_sc_agg(N, E, D, chunk, nbuf, with_deg):
    NW = _NC * _NS
    epw = E // NW
    nchunk = epw // chunk
    nfull = nchunk // nbuf
    rpt = (N // _NS) // 8 * 8
    tail = N - _NS * rpt

    mesh = plsc.VectorSubcoreMesh(core_axis_name="c", subcore_axis_name="s")

    out_type = [jax.ShapeDtypeStruct((_NC, N, D), jnp.float32)]
    scratch = [
        pltpu.VMEM((nchunk, chunk), jnp.int32),
        pltpu.VMEM((nchunk, chunk), jnp.int32),
        [pltpu.VMEM((chunk, D), jnp.float32) for _ in range(nbuf)],
        [pltpu.SemaphoreType.DMA for _ in range(nbuf)],
        [pltpu.SemaphoreType.DMA for _ in range(nbuf)],
        pltpu.VMEM_SHARED((N, D), jnp.float32),
    ]
    if with_deg:
        out_type.append(jax.ShapeDtypeStruct((_NC, N, 16), jnp.float32))
        scratch += [
            pltpu.VMEM((chunk, 16), jnp.float32),
            pltpu.SemaphoreType.DMA,
            pltpu.VMEM_SHARED((N, 16), jnp.float32),
        ]

    def body(p_hbm, src_hbm, dst_hbm, z_hbm, z16_hbm, *refs):
        if with_deg:
            (agg_out, deg_out, idx_s, idx_d, rows, gsem, ssem, agg_sp,
             ones, dsem, deg_sp) = refs
        else:
            agg_out, idx_s, idx_d, rows, gsem, ssem, agg_sp = refs
        c = lax.axis_index("c")
        s = lax.axis_index("s")
        wid = c * _NS + s

        pltpu.sync_copy(src_hbm.at[pl.ds(wid * nchunk, nchunk)], idx_s)
        pltpu.sync_copy(dst_hbm.at[pl.ds(wid * nchunk, nchunk)], idx_d)

        r0 = s * rpt
        pltpu.sync_copy(z_hbm.at[pl.ds(r0, rpt)], agg_sp.at[pl.ds(r0, rpt)])
        if with_deg:
            pltpu.sync_copy(z16_hbm.at[pl.ds(r0, rpt)],
                            deg_sp.at[pl.ds(r0, rpt)])
            for j in range(chunk):
                ones[j, :] = jnp.ones((16,), jnp.float32)
        if tail:
            t0 = _NS * rpt

            @pl.when(s == _NS - 1)
            def _():
                pltpu.sync_copy(z_hbm.at[pl.ds(t0, tail)],
                                agg_sp.at[pl.ds(t0, tail)])
                if with_deg:
                    pltpu.sync_copy(z16_hbm.at[pl.ds(t0, tail)],
                                    deg_sp.at[pl.ds(t0, tail)])
        plsc.subcore_barrier()

        def g_start(i, b):
            pltpu.async_copy(p_hbm.at[idx_s.at[i]], rows[b], gsem[b])

        def g_wait(b):
            pltpu.make_async_copy(p_hbm.at[idx_s.at[0]], rows[b],
                                  gsem[b]).wait()

        def s_start(i, b):
            pltpu.async_copy(rows[b], agg_sp.at[idx_d.at[i]], ssem[b],
                             add=True)
            if with_deg:
                pltpu.async_copy(ones, deg_sp.at[idx_d.at[i]], dsem,
                                 add=True)

        def s_wait(b):
            pltpu.make_async_copy(rows[b], agg_sp.at[idx_d.at[0]],
                                  ssem[b]).wait()
            if with_deg:
                pltpu.make_async_copy(ones, deg_sp.at[idx_d.at[0]],
                                      dsem).wait()

        for b in range(nbuf):
            g_start(b, b)

        def round_body(r, carry):
            i0 = r * nbuf
            for b in range(nbuf):
                g_wait(b)
                s_start(i0 + b, b)
            for b in range(nbuf):
                i_next = i0 + nbuf + b
                s_wait(b)

                @pl.when(i_next < nchunk)
                def _():
                    g_start(i_next, b)
            return carry

        lax.fori_loop(0, nfull, round_body, 0)
        for b in range(nchunk - nfull * nbuf):
            g_wait(b)
            s_start(nfull * nbuf + b, b)
            s_wait(b)
        plsc.subcore_barrier()

        pltpu.sync_copy(agg_sp.at[pl.ds(r0, rpt)],
                        agg_out.at[c, pl.ds(r0, rpt)])
        if with_deg:
            pltpu.sync_copy(deg_sp.at[pl.ds(r0, rpt)],
                            deg_out.at[c, pl.ds(r0, rpt)])
        if tail:
            t0 = _NS * rpt

            @pl.when(s == _NS - 1)
            def _():
                pltpu.sync_copy(agg_sp.at[pl.ds(t0, tail)],
                                agg_out.at[c, pl.ds(t0, tail)])
                if with_deg:
                    pltpu.sync_copy(deg_sp.at[pl.ds(t0, tail)],
                                    deg_out.at[c, pl.ds(t0, tail)])

    return pl.kernel(body, out_type=out_type, mesh=mesh,
                     scratch_types=scratch,
                     compiler_params=pltpu.CompilerParams(
                         use_tc_tiling_on_sc=False))


def _make_sc_deg(N, E, chunk, lag):
    NW = _NC * _NS
    epw = E // NW
    nchunk = epw // chunk
    rpt = (N // _NS) // 8 * 8
    tail = N - _NS * rpt

    mesh = plsc.VectorSubcoreMesh(core_axis_name="c", subcore_axis_name="s")

    out_type = jax.ShapeDtypeStruct((_NC, N, 16), jnp.float32)
    scratch = [
        pltpu.VMEM((nchunk, chunk), jnp.int32),
        pltpu.VMEM((chunk, 16), jnp.float32),
        pltpu.SemaphoreType.DMA,
        pltpu.VMEM_SHARED((N, 16), jnp.float32),
    ]

    def body(dst_hbm, z16_hbm, deg_out, idx_d, ones, dsem, deg_sp):
        c = lax.axis_index("c")
        s = lax.axis_index("s")
        wid = c * _NS + s

        pltpu.sync_copy(dst_hbm.at[pl.ds(wid * nchunk, nchunk)], idx_d)
        for j in range(chunk):
            ones[j, :] = jnp.ones((16,), jnp.float32)
        r0 = s * rpt
        pltpu.sync_copy(z16_hbm.at[pl.ds(r0, rpt)], deg_sp.at[pl.ds(r0, rpt)])
        if tail:
            t0 = _NS * rpt

            @pl.when(s == _NS - 1)
            def _():
                pltpu.sync_copy(z16_hbm.at[pl.ds(t0, tail)],
                                deg_sp.at[pl.ds(t0, tail)])
        plsc.subcore_barrier()

        def d_wait():
            pltpu.make_async_copy(ones, deg_sp.at[idx_d.at[0]], dsem).wait()

        def step(i, carry):
            pltpu.async_copy(ones, deg_sp.at[idx_d.at[i]], dsem, add=True)

            @pl.when(i >= lag)
            def _():
                d_wait()
            return carry

        lax.fori_loop(0, nchunk, step, 0)
        for _ in range(lag):
            d_wait()
        plsc.subcore_barrier()

        pltpu.sync_copy(deg_sp.at[pl.ds(r0, rpt)],
                        deg_out.at[c, pl.ds(r0, rpt)])
        if tail:
            t0 = _NS * rpt

            @pl.when(s == _NS - 1)
            def _():
                pltpu.sync_copy(deg_sp.at[pl.ds(t0, tail)],
                                deg_out.at[c, pl.ds(t0, tail)])

    return pl.kernel(body, out_type=out_type, mesh=mesh,
                     scratch_types=scratch,
                     compiler_params=pltpu.CompilerParams(
                         use_tc_tiling_on_sc=False))


def _mm_first(x_ref, ws_ref, wn_ref, ps_ref, pn_ref):
    x = x_ref[...]
    ps_ref[...] = jnp.dot(x, ws_ref[...], preferred_element_type=jnp.float32,
                          precision=lax.Precision.HIGHEST)
    pn_ref[...] = jnp.dot(x, wn_ref[...], preferred_element_type=jnp.float32,
                          precision=lax.Precision.HIGHEST)


def _combine(ps_ref, agg_ref, deg_ref, b_ref, h_ref):
    deg = deg_ref[0, :, 0:1] + deg_ref[1, :, 0:1]
    inv = 1.0 / jnp.maximum(deg, 1.0)
    h_ref[...] = ps_ref[...] + (agg_ref[0] + agg_ref[1]) * inv + b_ref[...]


def _combine_mm(ps_ref, agg_ref, deg_ref, b_ref, ws_ref, wn_ref,
                ps_out, pn_out):
    deg = deg_ref[0, :, 0:1] + deg_ref[1, :, 0:1]
    inv = 1.0 / jnp.maximum(deg, 1.0)
    h = ps_ref[...] + (agg_ref[0] + agg_ref[1]) * inv + b_ref[...]
    h = jnp.maximum(h, 0.0)
    ps_out[...] = jnp.dot(h, ws_ref[...], preferred_element_type=jnp.float32,
                          precision=lax.Precision.HIGHEST)
    pn_out[...] = jnp.dot(h, wn_ref[...], preferred_element_type=jnp.float32,
                          precision=lax.Precision.HIGHEST)


def _tc_first(x, ws, wn, blk):
    N, Din = x.shape
    Dout = ws.shape[1]
    return pl.pallas_call(
        _mm_first,
        grid=(N // blk,),
        in_specs=[
            pl.BlockSpec((blk, Din), lambda i: (i, 0)),
            pl.BlockSpec((Din, Dout), lambda i: (0, 0)),
            pl.BlockSpec((Din, Dout), lambda i: (0, 0)),
        ],
        out_specs=[
            pl.BlockSpec((blk, Dout), lambda i: (i, 0)),
            pl.BlockSpec((blk, Dout), lambda i: (i, 0)),
        ],
        out_shape=[jax.ShapeDtypeStruct((N, Dout), jnp.float32)] * 2,
    )(x, ws, wn)


def _tc_combine_mm(ps, agg, deg, b, ws, wn, blk):
    N, Dh = ps.shape
    Dout = ws.shape[1]
    return pl.pallas_call(
        _combine_mm,
        grid=(N // blk,),
        in_specs=[
            pl.BlockSpec((blk, Dh), lambda i: (i, 0)),
            pl.BlockSpec((2, blk, Dh), lambda i: (0, i, 0)),
            pl.BlockSpec((2, blk, 16), lambda i: (0, i, 0)),
            pl.BlockSpec((1, Dh), lambda i: (0, 0)),
            pl.BlockSpec((Dh, Dout), lambda i: (0, 0)),
            pl.BlockSpec((Dh, Dout), lambda i: (0, 0)),
        ],
        out_specs=[
            pl.BlockSpec((blk, Dout), lambda i: (i, 0)),
            pl.BlockSpec((blk, Dout), lambda i: (i, 0)),
        ],
        out_shape=[jax.ShapeDtypeStruct((N, Dout), jnp.float32)] * 2,
    )(ps, agg, deg, b, ws, wn)


def _tc_combine(ps, agg, deg, b, blk):
    N, Dh = ps.shape
    return pl.pallas_call(
        _combine,
        grid=(N // blk,),
        in_specs=[
            pl.BlockSpec((blk, Dh), lambda i: (i, 0)),
            pl.BlockSpec((2, blk, Dh), lambda i: (0, i, 0)),
            pl.BlockSpec((2, blk, 16), lambda i: (0, i, 0)),
            pl.BlockSpec((1, Dh), lambda i: (0, 0)),
        ],
        out_specs=pl.BlockSpec((blk, Dh), lambda i: (i, 0)),
        out_shape=jax.ShapeDtypeStruct((N, Dh), jnp.float32),
    )(ps, agg, deg, b)


@jax.jit
def kernel(x, edge_index, W_self0, W_neigh0, b0, W_self1, W_neigh1, b1,
           W_self2, W_neigh2, b2):
    N, _ = x.shape
    E = edge_index.shape[1]
    D_hid = W_self0.shape[1]
    D_out = W_self2.shape[1]
    blk = 2000

    src = edge_index[0].reshape(E // _CHUNK, _CHUNK)
    dst = edge_index[1].reshape(E // _CHUNK, _CHUNK)
    z_h = jnp.zeros((N, D_hid), jnp.float32)
    z_o = jnp.zeros((N, D_out), jnp.float32)
    z16 = jnp.zeros((N, 16), jnp.float32)

    agg_h = _make_sc_agg(N, E, D_hid, _CHUNK, 3, False)
    agg_o = _make_sc_agg(N, E, D_out, _CHUNK, 6, False)
    deg_k = _make_sc_deg(N, E, _CHUNK, 8)

    deg = deg_k(dst, z16)
    ps0, pn0 = _tc_first(x, W_self0, W_neigh0, blk)
    (agg0,) = agg_h(pn0, src, dst, z_h, z16)
    ps1, pn1 = _tc_combine_mm(ps0, agg0, deg, b0.reshape(1, -1),
                              W_self1, W_neigh1, blk)
    (agg1,) = agg_h(pn1, src, dst, z_h, z16)
    ps2, pn2 = _tc_combine_mm(ps1, agg1, deg, b1.reshape(1, -1),
                              W_self2, W_neigh2, blk)
    (agg2,) = agg_o(pn2, src, dst, z_o, z16)
    return _tc_combine(ps2, agg2, deg, b2.reshape(1, -1), blk)


# speedup vs baseline: 1.1812x; 1.0490x over previous
import functools

import jax
import jax.numpy as jnp
from jax import lax
from jax.experimental import pallas as pl
from jax.experimental.pallas import tpu as pltpu
from jax.experimental.pallas import tpu_sc as plsc

_NC = 2
_NS = 16
_CHUNK = 80


def _make_sc_agg(N, E, D, chunk, nbuf, with_deg):
    NW = _NC * _NS
    epw = E // NW
    nchunk = epw // chunk
    nfull = nchunk // nbuf
    rpt = (N // _NS) // 8 * 8
    tail = N - _NS * rpt

    mesh = plsc.VectorSubcoreMesh(core_axis_name="c", subcore_axis_name="s")

    out_type = [jax.ShapeDtypeStruct((_NC, N, D), jnp.float32)]
    scratch = [
        pltpu.VMEM((nchunk, chunk), jnp.int32),
        pltpu.VMEM((nchunk, chunk), jnp.int32),
        [pltpu.VMEM((chunk, D), jnp.float32) for _ in range(nbuf)],
        [pltpu.SemaphoreType.DMA for _ in range(nbuf)],
        [pltpu.SemaphoreType.DMA for _ in range(nbuf)],
        pltpu.VMEM_SHARED((N, D), jnp.float32),
    ]
    if with_deg:
        out_type.append(jax.ShapeDtypeStruct((_NC, N, 16), jnp.float32))
        scratch += [
            pltpu.VMEM((chunk, 16), jnp.float32),
            pltpu.SemaphoreType.DMA,
            pltpu.VMEM_SHARED((N, 16), jnp.float32),
        ]

    def body(p_hbm, src_hbm, dst_hbm, z_hbm, z16_hbm, *refs):
        if with_deg:
            (agg_out, deg_out, idx_s, idx_d, rows, gsem, ssem, agg_sp,
             ones, dsem, deg_sp) = refs
        else:
            agg_out, idx_s, idx_d, rows, gsem, ssem, agg_sp = refs
        c = lax.axis_index("c")
        s = lax.axis_index("s")
        wid = c * _NS + s

        pltpu.sync_copy(src_hbm.at[pl.ds(wid * nchunk, nchunk)], idx_s)
        pltpu.sync_copy(dst_hbm.at[pl.ds(wid * nchunk, nchunk)], idx_d)

        r0 = s * rpt
        pltpu.sync_copy(z_hbm.at[pl.ds(r0, rpt)], agg_sp.at[pl.ds(r0, rpt)])
        if with_deg:
            pltpu.sync_copy(z16_hbm.at[pl.ds(r0, rpt)],
                            deg_sp.at[pl.ds(r0, rpt)])
            for j in range(chunk):
                ones[j, :] = jnp.ones((16,), jnp.float32)
        if tail:
            t0 = _NS * rpt

            @pl.when(s == _NS - 1)
            def _():
                pltpu.sync_copy(z_hbm.at[pl.ds(t0, tail)],
                                agg_sp.at[pl.ds(t0, tail)])
                if with_deg:
                    pltpu.sync_copy(z16_hbm.at[pl.ds(t0, tail)],
                                    deg_sp.at[pl.ds(t0, tail)])
        plsc.subcore_barrier()

        def g_start(i, b):
            pltpu.async_copy(p_hbm.at[idx_s.at[i]], rows[b], gsem[b])

        def g_wait(b):
            pltpu.make_async_copy(p_hbm.at[idx_s.at[0]], rows[b],
                                  gsem[b]).wait()

        def s_start(i, b):
            pltpu.async_copy(rows[b], agg_sp.at[idx_d.at[i]], ssem[b],
                             add=True)
            if with_deg:
                pltpu.async_copy(ones, deg_sp.at[idx_d.at[i]], dsem,
                                 add=True)

        def s_wait(b):
            pltpu.make_async_copy(rows[b], agg_sp.at[idx_d.at[0]],
                                  ssem[b]).wait()
            if with_deg:
                pltpu.make_async_copy(ones, deg_sp.at[idx_d.at[0]],
                                      dsem).wait()

        for b in range(nbuf):
            g_start(b, b)

        def round_body(r, carry):
            i0 = r * nbuf
            for b in range(nbuf):
                g_wait(b)
                s_start(i0 + b, b)
            for b in range(nbuf):
                i_next = i0 + nbuf + b
                s_wait(b)

                @pl.when(i_next < nchunk)
                def _():
                    g_start(i_next, b)
            return carry

        lax.fori_loop(0, nfull, round_body, 0)
        for b in range(nchunk - nfull * nbuf):
            g_wait(b)
            s_start(nfull * nbuf + b, b)
            s_wait(b)
        plsc.subcore_barrier()

        pltpu.sync_copy(agg_sp.at[pl.ds(r0, rpt)],
                        agg_out.at[c, pl.ds(r0, rpt)])
        if with_deg:
            pltpu.sync_copy(deg_sp.at[pl.ds(r0, rpt)],
                            deg_out.at[c, pl.ds(r0, rpt)])
        if tail:
            t0 = _NS * rpt

            @pl.when(s == _NS - 1)
            def _():
                pltpu.sync_copy(agg_sp.at[pl.ds(t0, tail)],
                                agg_out.at[c, pl.ds(t0, tail)])
                if with_deg:
                    pltpu.sync_copy(deg_sp.at[pl.ds(t0, tail)],
                                    deg_out.at[c, pl.ds(t0, tail)])

    return pl.kernel(body, out_type=out_type, mesh=mesh,
                     scratch_types=scratch,
                     compiler_params=pltpu.CompilerParams(
                         use_tc_tiling_on_sc=False))


def _make_sc_agg_ring(N, E, D, chunk, nbuf):
    NW = _NC * _NS
    epw = E // NW
    nchunk = epw // chunk
    nfull = nchunk // nbuf
    rem = nchunk - nfull * nbuf
    rpt = (N // _NS) // 8 * 8
    tail = N - _NS * rpt

    mesh = plsc.VectorSubcoreMesh(core_axis_name="c", subcore_axis_name="s")

    out_type = jax.ShapeDtypeStruct((_NC, N, D), jnp.float32)
    scratch = [
        [pltpu.VMEM((chunk,), jnp.int32) for _ in range(nbuf)],
        [pltpu.VMEM((chunk,), jnp.int32) for _ in range(nbuf)],
        [pltpu.VMEM((chunk, D), jnp.float32) for _ in range(nbuf)],
        [pltpu.SemaphoreType.DMA for _ in range(nbuf)],
        [pltpu.SemaphoreType.DMA for _ in range(nbuf)],
        [pltpu.SemaphoreType.DMA for _ in range(nbuf)],
        [pltpu.SemaphoreType.DMA for _ in range(nbuf)],
        pltpu.VMEM_SHARED((N, D), jnp.float32),
    ]

    def body(p_hbm, src_hbm, dst_hbm, z_hbm, agg_out,
             idx_s, idx_d, rows, isem, dsem, gsem, ssem, agg_sp):
        c = lax.axis_index("c")
        s = lax.axis_index("s")
        wid = c * _NS + s
        row0 = wid * nchunk

        r0 = s * rpt
        pltpu.sync_copy(z_hbm.at[pl.ds(r0, rpt)], agg_sp.at[pl.ds(r0, rpt)])
        if tail:
            t0 = _NS * rpt

            @pl.when(s == _NS - 1)
            def _():
                pltpu.sync_copy(z_hbm.at[pl.ds(t0, tail)],
                                agg_sp.at[pl.ds(t0, tail)])
        plsc.subcore_barrier()

        def is_load(i, b):
            pltpu.async_copy(src_hbm.at[row0 + i], idx_s[b], isem[b])

        def is_wait(b):
            pltpu.make_async_copy(src_hbm.at[row0], idx_s[b], isem[b]).wait()

        def id_load(i, b):
            pltpu.async_copy(dst_hbm.at[row0 + i], idx_d[b], dsem[b])

        def id_wait(b):
            pltpu.make_async_copy(dst_hbm.at[row0], idx_d[b], dsem[b]).wait()

        def g_start(b):
            pltpu.async_copy(p_hbm.at[idx_s[b]], rows[b], gsem[b])

        def g_wait(b):
            pltpu.make_async_copy(p_hbm.at[idx_s[b]], rows[b], gsem[b]).wait()

        def s_start(b):
            pltpu.async_copy(rows[b], agg_sp.at[idx_d[b]], ssem[b], add=True)

        def s_wait(b):
            pltpu.make_async_copy(rows[b], agg_sp.at[idx_d[b]],
                                  ssem[b]).wait()

        for b in range(nbuf):
            is_load(b, b)
            id_load(b, b)
        for b in range(nbuf):
            is_wait(b)
            g_start(b)

        def round_body(r, carry):
            i0 = r * nbuf
            for b in range(nbuf):
                id_wait(b)
                g_wait(b)
                s_start(b)
                inext = i0 + nbuf + b

                @pl.when(inext < nchunk)
                def _():
                    is_load(inext, b)
            for b in range(nbuf):
                inext = i0 + nbuf + b
                s_wait(b)

                @pl.when(inext < nchunk)
                def _():
                    id_load(inext, b)
                    is_wait(b)
                    g_start(b)
            return carry

        lax.fori_loop(0, nfull, round_body, 0)
        for b in range(rem):
            id_wait(b)
            g_wait(b)
            s_start(b)
            s_wait(b)
        plsc.subcore_barrier()

        pltpu.sync_copy(agg_sp.at[pl.ds(r0, rpt)],
                        agg_out.at[c, pl.ds(r0, rpt)])
        if tail:
            t0 = _NS * rpt

            @pl.when(s == _NS - 1)
            def _():
                pltpu.sync_copy(agg_sp.at[pl.ds(t0, tail)],
                                agg_out.at[c, pl.ds(t0, tail)])

    return pl.kernel(body, out_type=out_type, mesh=mesh,
                     scratch_types=scratch,
                     compiler_params=pltpu.CompilerParams(
                         use_tc_tiling_on_sc=False))


def _make_sc_deg(N, E, chunk, lag):
    NW = _NC * _NS
    epw = E // NW
    nchunk = epw // chunk
    rpt = (N // _NS) // 8 * 8
    tail = N - _NS * rpt

    mesh = plsc.VectorSubcoreMesh(core_axis_name="c", subcore_axis_name="s")

    out_type = jax.ShapeDtypeStruct((_NC, N, 16), jnp.float32)
    scratch = [
        pltpu.VMEM((nchunk, chunk), jnp.int32),
        pltpu.VMEM((chunk, 16), jnp.float32),
        pltpu.SemaphoreType.DMA,
        pltpu.VMEM_SHARED((N, 16), jnp.float32),
    ]

    def body(dst_hbm, z16_hbm, deg_out, idx_d, ones, dsem, deg_sp):
        c = lax.axis_index("c")
        s = lax.axis_index("s")
        wid = c * _NS + s

        pltpu.sync_copy(dst_hbm.at[pl.ds(wid * nchunk, nchunk)], idx_d)
        for j in range(chunk):
            ones[j, :] = jnp.ones((16,), jnp.float32)
        r0 = s * rpt
        pltpu.sync_copy(z16_hbm.at[pl.ds(r0, rpt)], deg_sp.at[pl.ds(r0, rpt)])
        if tail:
            t0 = _NS * rpt

            @pl.when(s == _NS - 1)
            def _():
                pltpu.sync_copy(z16_hbm.at[pl.ds(t0, tail)],
                                deg_sp.at[pl.ds(t0, tail)])
        plsc.subcore_barrier()

        def d_wait():
            pltpu.make_async_copy(ones, deg_sp.at[idx_d.at[0]], dsem).wait()

        def step(i, carry):
            pltpu.async_copy(ones, deg_sp.at[idx_d.at[i]], dsem, add=True)

            @pl.when(i >= lag)
            def _():
                d_wait()
            return carry

        lax.fori_loop(0, nchunk, step, 0)
        for _ in range(lag):
            d_wait()
        plsc.subcore_barrier()

        pltpu.sync_copy(deg_sp.at[pl.ds(r0, rpt)],
                        deg_out.at[c, pl.ds(r0, rpt)])
        if tail:
            t0 = _NS * rpt

            @pl.when(s == _NS - 1)
            def _():
                pltpu.sync_copy(deg_sp.at[pl.ds(t0, tail)],
                                deg_out.at[c, pl.ds(t0, tail)])

    return pl.kernel(body, out_type=out_type, mesh=mesh,
                     scratch_types=scratch,
                     compiler_params=pltpu.CompilerParams(
                         use_tc_tiling_on_sc=False))


def _mm_first(x_ref, ws_ref, wn_ref, ps_ref, pn_ref):
    x = x_ref[...]
    ps_ref[...] = jnp.dot(x, ws_ref[...], preferred_element_type=jnp.float32,
                          precision=lax.Precision.HIGHEST)
    pn_ref[...] = jnp.dot(x, wn_ref[...], preferred_element_type=jnp.float32,
                          precision=lax.Precision.HIGHEST)


def _combine(ps_ref, agg_ref, deg_ref, b_ref, h_ref):
    deg = deg_ref[0, :, 0:1] + deg_ref[1, :, 0:1]
    inv = 1.0 / jnp.maximum(deg, 1.0)
    h_ref[...] = ps_ref[...] + (agg_ref[0] + agg_ref[1]) * inv + b_ref[...]


def _combine_mm(ps_ref, agg_ref, deg_ref, b_ref, ws_ref, wn_ref,
                ps_out, pn_out):
    deg = deg_ref[0, :, 0:1] + deg_ref[1, :, 0:1]
    inv = 1.0 / jnp.maximum(deg, 1.0)
    h = ps_ref[...] + (agg_ref[0] + agg_ref[1]) * inv + b_ref[...]
    h = jnp.maximum(h, 0.0)
    ps_out[...] = jnp.dot(h, ws_ref[...], preferred_element_type=jnp.float32,
                          precision=lax.Precision.HIGHEST)
    pn_out[...] = jnp.dot(h, wn_ref[...], preferred_element_type=jnp.float32,
                          precision=lax.Precision.HIGHEST)


def _tc_first(x, ws, wn, blk):
    N, Din = x.shape
    Dout = ws.shape[1]
    return pl.pallas_call(
        _mm_first,
        grid=(N // blk,),
        in_specs=[
            pl.BlockSpec((blk, Din), lambda i: (i, 0)),
            pl.BlockSpec((Din, Dout), lambda i: (0, 0)),
            pl.BlockSpec((Din, Dout), lambda i: (0, 0)),
        ],
        out_specs=[
            pl.BlockSpec((blk, Dout), lambda i: (i, 0)),
            pl.BlockSpec((blk, Dout), lambda i: (i, 0)),
        ],
        out_shape=[jax.ShapeDtypeStruct((N, Dout), jnp.float32)] * 2,
    )(x, ws, wn)


def _tc_combine_mm(ps, agg, deg, b, ws, wn, blk):
    N, Dh = ps.shape
    Dout = ws.shape[1]
    return pl.pallas_call(
        _combine_mm,
        grid=(N // blk,),
        in_specs=[
            pl.BlockSpec((blk, Dh), lambda i: (i, 0)),
            pl.BlockSpec((2, blk, Dh), lambda i: (0, i, 0)),
            pl.BlockSpec((2, blk, 16), lambda i: (0, i, 0)),
            pl.BlockSpec((1, Dh), lambda i: (0, 0)),
            pl.BlockSpec((Dh, Dout), lambda i: (0, 0)),
            pl.BlockSpec((Dh, Dout), lambda i: (0, 0)),
        ],
        out_specs=[
            pl.BlockSpec((blk, Dout), lambda i: (i, 0)),
            pl.BlockSpec((blk, Dout), lambda i: (i, 0)),
        ],
        out_shape=[jax.ShapeDtypeStruct((N, Dout), jnp.float32)] * 2,
    )(ps, agg, deg, b, ws, wn)


def _tc_combine(ps, agg, deg, b, blk):
    N, Dh = ps.shape
    return pl.pallas_call(
        _combine,
        grid=(N // blk,),
        in_specs=[
            pl.BlockSpec((blk, Dh), lambda i: (i, 0)),
            pl.BlockSpec((2, blk, Dh), lambda i: (0, i, 0)),
            pl.BlockSpec((2, blk, 16), lambda i: (0, i, 0)),
            pl.BlockSpec((1, Dh), lambda i: (0, 0)),
        ],
        out_specs=pl.BlockSpec((blk, Dh), lambda i: (i, 0)),
        out_shape=jax.ShapeDtypeStruct((N, Dh), jnp.float32),
    )(ps, agg, deg, b)


@jax.jit
def kernel(x, edge_index, W_self0, W_neigh0, b0, W_self1, W_neigh1, b1,
           W_self2, W_neigh2, b2):
    N, _ = x.shape
    E = edge_index.shape[1]
    D_hid = W_self0.shape[1]
    D_out = W_self2.shape[1]
    blk = 2000

    src = edge_index[0].reshape(E // _CHUNK, _CHUNK)
    dst = edge_index[1].reshape(E // _CHUNK, _CHUNK)
    z_h = jnp.zeros((N, D_hid), jnp.float32)
    z_o = jnp.zeros((N, D_out), jnp.float32)
    z16 = jnp.zeros((N, 16), jnp.float32)

    agg_h = _make_sc_agg_ring(N, E, D_hid, _CHUNK, 4)
    agg_o = _make_sc_agg_ring(N, E, D_out, _CHUNK, 6)
    deg_k = _make_sc_deg(N, E, _CHUNK, 8)

    deg = deg_k(dst, z16)
    ps0, pn0 = _tc_first(x, W_self0, W_neigh0, blk)
    agg0 = agg_h(pn0, src, dst, z_h)
    ps1, pn1 = _tc_combine_mm(ps0, agg0, deg, b0.reshape(1, -1),
                              W_self1, W_neigh1, blk)
    agg1 = agg_h(pn1, src, dst, z_h)
    ps2, pn2 = _tc_combine_mm(ps1, agg1, deg, b1.reshape(1, -1),
                              W_self2, W_neigh2, blk)
    agg2 = agg_o(pn2, src, dst, z_o)
    return _tc_combine(ps2, agg2, deg, b2.reshape(1, -1), blk)
